# baseline (speedup 1.0000x reference)
import functools

import jax
import jax.numpy as jnp
from jax import lax
from jax.experimental import pallas as pl
from jax.experimental.pallas import tpu as pltpu
from jax.experimental.pallas import tpu_sc as plsc

NUM_CORES = 2
NUM_SUBCORES = 16
NW = NUM_CORES * NUM_SUBCORES
CH = 32
SUB = 8


def _window(emb, mem, q, r, bp):
    b = emb.shape[0]
    m = mem.shape[0]
    w = jnp.zeros((bp,) + emb.shape[1:], emb.dtype)
    w = lax.dynamic_update_slice(w, emb, (r,) + (0,) * (emb.ndim - 1))
    i8 = jnp.arange(SUB).reshape((SUB,) + (1,) * (emb.ndim - 1))
    head = lax.dynamic_slice_in_dim(mem, q, SUB, axis=0)
    w = w.at[0:SUB].set(jnp.where(i8 < r, head, w[0:SUB]))
    tail = mem[(q + b + jnp.arange(SUB)) % m]
    w = w.at[b:bp].set(jnp.where(i8 >= r, tail, w[b:bp]))
    return w


def kernel(mem_img, mem_gps, mem_coords, img_emb, gps_emb, gps_coords, ptr):
    M, D = mem_img.shape
    B = img_emb.shape[0]
    C = mem_coords.shape[1]
    W = 2 * D + C
    CP = 128
    BP = B + SUB
    rows_per_w = M // NW
    n_chunks = rows_per_w // CH
    n_pairs = n_chunks // 2

    p = jnp.asarray(ptr, jnp.int32) % jnp.int32(M)
    q = p & jnp.int32(-SUB)
    r = p & jnp.int32(SUB - 1)
    q_vec = jnp.full((16,), q, dtype=jnp.int32)

    ie2 = _window(img_emb, mem_img, q, r, BP)
    ge2 = _window(gps_emb, mem_gps, q, r, BP)
    gc2 = jnp.pad(_window(gps_coords, mem_coords, q, r, BP),
                  ((0, 0), (0, CP - C)))
    mc_u = jnp.pad(mem_coords, ((0, 0), (0, CP - C)))

    mesh = plsc.VectorSubcoreMesh(core_axis_name="c", subcore_axis_name="s")

    @functools.partial(
        pl.kernel,
        out_type=(jax.ShapeDtypeStruct((M, 2 * D), jnp.float32),
                  jax.ShapeDtypeStruct((M, CP), jnp.float32)),
        mesh=mesh,
        scratch_types=[
            pltpu.VMEM((CH, D), jnp.float32),
            pltpu.VMEM((CH, D), jnp.float32),
            pltpu.VMEM((CH, CP), jnp.float32),
            pltpu.VMEM((CH, D), jnp.float32),
            pltpu.VMEM((CH, D), jnp.float32),
            pltpu.VMEM((CH, CP), jnp.float32),
            pltpu.VMEM((16,), jnp.int32),
            pltpu.SemaphoreType.DMA,
            pltpu.SemaphoreType.DMA,
            pltpu.SemaphoreType.DMA,
            pltpu.SemaphoreType.DMA,
        ],
    )
    def run(mi, mg, mc, ie, ge, gc, qv, out, ocrd,
            img0, gps0, crd0, img1, gps1, crd1, qbuf, r0, r1, w0, w1):
        wid = lax.axis_index("s") * NUM_CORES + lax.axis_index("c")
        base = pl.multiple_of(wid * rows_per_w, rows_per_w)
        pltpu.sync_copy(qv, qbuf)
        qk = qbuf[...][0]

        def read(c0, bi, bg, bc, sem):
            d = c0 - qk
            s = jnp.where(d < 0, d + M, d)
            full_in = s <= BP - CH

            @pl.when(full_in)
            def _():
                sa = pl.multiple_of(s, SUB)
                pltpu.async_copy(ie.at[pl.ds(sa, CH)], bi, sem)
                pltpu.async_copy(ge.at[pl.ds(sa, CH)], bg, sem)
                pltpu.async_copy(gc.at[pl.ds(sa, CH)], bc, sem)

            @pl.when(jnp.logical_not(full_in))
            def _():
                pltpu.async_copy(mi.at[pl.ds(c0, CH)], bi, sem)
                pltpu.async_copy(mg.at[pl.ds(c0, CH)], bg, sem)
                pltpu.async_copy(mc.at[pl.ds(c0, CH)], bc, sem)

        def wait_read(bi, bg, bc, sem):
            pltpu.make_async_copy(mi.at[pl.ds(0, CH)], bi, sem).wait()
            pltpu.make_async_copy(mg.at[pl.ds(0, CH)], bg, sem).wait()
            pltpu.make_async_copy(mc.at[pl.ds(0, CH)], bc, sem).wait()

        def overlay(c0, bi, bg, bc):
            d = c0 - qk
            s = jnp.where(d < 0, d + M, d)
            full_in = s <= BP - CH
            for k in range(CH // SUB):
                e = s + k * SUB
                e = jnp.where(e >= M, e - M, e)

                @pl.when(jnp.logical_and(jnp.logical_not(full_in), e < BP))
                def _ov(e=e, k=k):
                    ea = pl.multiple_of(e, SUB)
                    pltpu.sync_copy(ie.at[pl.ds(ea, SUB)],
                                    bi.at[pl.ds(k * SUB, SUB)])
                    pltpu.sync_copy(ge.at[pl.ds(ea, SUB)],
                                    bg.at[pl.ds(k * SUB, SUB)])
                    pltpu.sync_copy(gc.at[pl.ds(ea, SUB)],
                                    bc.at[pl.ds(k * SUB, SUB)])

        def write(c0, bi, bg, bc, sem):
            pltpu.async_copy(bi, out.at[pl.ds(c0, CH), pl.ds(0, D)], sem)
            pltpu.async_copy(bg, out.at[pl.ds(c0, CH), pl.ds(D, D)], sem)
            pltpu.async_copy(bc, ocrd.at[pl.ds(c0, CH)], sem)

        def wait_write(bi, bg, bc, sem):
            pltpu.make_async_copy(bi, out.at[pl.ds(0, CH), pl.ds(0, D)],
                                  sem).wait()
            pltpu.make_async_copy(bg, out.at[pl.ds(0, CH), pl.ds(D, D)],
                                  sem).wait()
            pltpu.make_async_copy(bc, ocrd.at[pl.ds(0, CH)], sem).wait()

        read(base, img0, gps0, crd0, r0)

        def pair_body(t2, carry):
            a = pl.multiple_of(base + (2 * t2) * CH, CH)
            b = pl.multiple_of(base + (2 * t2 + 1) * CH, CH)

            @pl.when(t2 > 0)
            def _():
                wait_write(img1, gps1, crd1, w1)

            read(b, img1, gps1, crd1, r1)
            wait_read(img0, gps0, crd0, r0)
            overlay(a, img0, gps0, crd0)
            write(a, img0, gps0, crd0, w0)
            wait_read(img1, gps1, crd1, r1)
            overlay(b, img1, gps1, crd1)
            write(b, img1, gps1, crd1, w1)

            @pl.when(t2 < n_pairs - 1)
            def _():
                wait_write(img0, gps0, crd0, w0)
                read(pl.multiple_of(base + (2 * t2 + 2) * CH, CH),
                     img0, gps0, crd0, r0)

            return carry

        lax.fori_loop(0, n_pairs, pair_body, 0)
        wait_write(img0, gps0, crd0, w0)
        wait_write(img1, gps1, crd1, w1)

    out_sc, out_crd = run(mem_img, mem_gps, mc_u, ie2, ge2, gc2, q_vec)

    return jnp.concatenate([out_sc, out_crd[:, :C]], axis=1)


# speedup vs baseline: 1.0781x; 1.0781x over previous
import functools

import jax
import jax.numpy as jnp
from jax import lax
from jax.experimental import pallas as pl
from jax.experimental.pallas import tpu as pltpu
from jax.experimental.pallas import tpu_sc as plsc

NUM_CORES = 2
NUM_SUBCORES = 16
NW = NUM_CORES * NUM_SUBCORES
CH = 32
SUB = 8


def _window(emb, mem, q, r, bp):
    b = emb.shape[0]
    m = mem.shape[0]
    w = jnp.zeros((bp,) + emb.shape[1:], emb.dtype)
    w = lax.dynamic_update_slice(w, emb, (r,) + (0,) * (emb.ndim - 1))
    i8 = jnp.arange(SUB).reshape((SUB,) + (1,) * (emb.ndim - 1))
    head = lax.dynamic_slice_in_dim(mem, q, SUB, axis=0)
    w = w.at[0:SUB].set(jnp.where(i8 < r, head, w[0:SUB]))
    tail = mem[(q + b + jnp.arange(SUB)) % m]
    w = w.at[b:bp].set(jnp.where(i8 >= r, tail, w[b:bp]))
    return w


def kernel(mem_img, mem_gps, mem_coords, img_emb, gps_emb, gps_coords, ptr):
    M, D = mem_img.shape
    B = img_emb.shape[0]
    C = mem_coords.shape[1]
    W = 2 * D + C
    CP = 128
    BP = B + SUB
    rows_per_w = M // NW
    n_chunks = rows_per_w // CH
    n_pairs = n_chunks // 2

    p = jnp.asarray(ptr, jnp.int32) % jnp.int32(M)
    q = p & jnp.int32(-SUB)
    r = p & jnp.int32(SUB - 1)
    q_vec = jnp.full((16,), q, dtype=jnp.int32)

    ie2 = _window(img_emb, mem_img, q, r, BP)
    ge2 = _window(gps_emb, mem_gps, q, r, BP)
    gc2 = jnp.pad(_window(gps_coords, mem_coords, q, r, BP),
                  ((0, 0), (0, CP - C)))
    mc_u = jnp.pad(mem_coords, ((0, 0), (0, CP - C)))

    mesh = plsc.VectorSubcoreMesh(core_axis_name="c", subcore_axis_name="s")

    @functools.partial(
        pl.kernel,
        out_type=(jax.ShapeDtypeStruct((M, W), jnp.float32),
                  jax.ShapeDtypeStruct((M, CP), jnp.float32)),
        mesh=mesh,
        scratch_types=[
            pltpu.VMEM((CH, D), jnp.float32),
            pltpu.VMEM((CH, D), jnp.float32),
            pltpu.VMEM((CH, CP), jnp.float32),
            pltpu.VMEM((CH, D), jnp.float32),
            pltpu.VMEM((CH, D), jnp.float32),
            pltpu.VMEM((CH, CP), jnp.float32),
            pltpu.VMEM((16,), jnp.int32),
            pltpu.SemaphoreType.DMA,
            pltpu.SemaphoreType.DMA,
            pltpu.SemaphoreType.DMA,
            pltpu.SemaphoreType.DMA,
        ],
    )
    def run(mi, mg, mc, ie, ge, gc, qv, out, ocrd,
            img0, gps0, crd0, img1, gps1, crd1, qbuf, r0, r1, w0, w1):
        wid = lax.axis_index("s") * NUM_CORES + lax.axis_index("c")
        base = pl.multiple_of(wid * rows_per_w, rows_per_w)
        pltpu.sync_copy(qv, qbuf)
        qk = qbuf[...][0]

        def read(c0, bi, bg, bc, sem):
            d = c0 - qk
            s = jnp.where(d < 0, d + M, d)
            full_in = s <= BP - CH

            @pl.when(full_in)
            def _():
                sa = pl.multiple_of(s, SUB)
                pltpu.async_copy(ie.at[pl.ds(sa, CH)], bi, sem)
                pltpu.async_copy(ge.at[pl.ds(sa, CH)], bg, sem)
                pltpu.async_copy(gc.at[pl.ds(sa, CH)], bc, sem)

            @pl.when(jnp.logical_not(full_in))
            def _():
                pltpu.async_copy(mi.at[pl.ds(c0, CH)], bi, sem)
                pltpu.async_copy(mg.at[pl.ds(c0, CH)], bg, sem)
                pltpu.async_copy(mc.at[pl.ds(c0, CH)], bc, sem)

        def wait_read(bi, bg, bc, sem):
            pltpu.make_async_copy(mi.at[pl.ds(0, CH)], bi, sem).wait()
            pltpu.make_async_copy(mg.at[pl.ds(0, CH)], bg, sem).wait()
            pltpu.make_async_copy(mc.at[pl.ds(0, CH)], bc, sem).wait()

        def overlay(c0, bi, bg, bc):
            d = c0 - qk
            s = jnp.where(d < 0, d + M, d)
            full_in = s <= BP - CH
            for k in range(CH // SUB):
                e = s + k * SUB
                e = jnp.where(e >= M, e - M, e)

                @pl.when(jnp.logical_and(jnp.logical_not(full_in), e < BP))
                def _ov(e=e, k=k):
                    ea = pl.multiple_of(e, SUB)
                    pltpu.sync_copy(ie.at[pl.ds(ea, SUB)],
                                    bi.at[pl.ds(k * SUB, SUB)])
                    pltpu.sync_copy(ge.at[pl.ds(ea, SUB)],
                                    bg.at[pl.ds(k * SUB, SUB)])
                    pltpu.sync_copy(gc.at[pl.ds(ea, SUB)],
                                    bc.at[pl.ds(k * SUB, SUB)])

        def write(c0, bi, bg, bc, sem):
            pltpu.async_copy(bi, out.at[pl.ds(c0, CH), pl.ds(0, D)], sem)
            pltpu.async_copy(bg, out.at[pl.ds(c0, CH), pl.ds(D, D)], sem)
            pltpu.async_copy(bc, ocrd.at[pl.ds(c0, CH)], sem)

        def wait_write(bi, bg, bc, sem):
            pltpu.make_async_copy(bi, out.at[pl.ds(0, CH), pl.ds(0, D)],
                                  sem).wait()
            pltpu.make_async_copy(bg, out.at[pl.ds(0, CH), pl.ds(D, D)],
                                  sem).wait()
            pltpu.make_async_copy(bc, ocrd.at[pl.ds(0, CH)], sem).wait()

        read(base, img0, gps0, crd0, r0)

        def pair_body(t2, carry):
            a = pl.multiple_of(base + (2 * t2) * CH, CH)
            b = pl.multiple_of(base + (2 * t2 + 1) * CH, CH)

            @pl.when(t2 > 0)
            def _():
                wait_write(img1, gps1, crd1, w1)

            read(b, img1, gps1, crd1, r1)
            wait_read(img0, gps0, crd0, r0)
            overlay(a, img0, gps0, crd0)
            write(a, img0, gps0, crd0, w0)
            wait_read(img1, gps1, crd1, r1)
            overlay(b, img1, gps1, crd1)
            write(b, img1, gps1, crd1, w1)

            @pl.when(t2 < n_pairs - 1)
            def _():
                wait_write(img0, gps0, crd0, w0)
                read(pl.multiple_of(base + (2 * t2 + 2) * CH, CH),
                     img0, gps0, crd0, r0)

            return carry

        lax.fori_loop(0, n_pairs, pair_body, 0)
        wait_write(img0, gps0, crd0, w0)
        wait_write(img1, gps1, crd1, w1)

    out_sc, out_crd = run(mem_img, mem_gps, mc_u, ie2, ge2, gc2, q_vec)

    BRT = 512

    def tc_body(_, crd_ref, o_ref):
        o_ref[...] = crd_ref[...]

    return pl.pallas_call(
        tc_body,
        grid=(M // BRT,),
        in_specs=[
            pl.BlockSpec(memory_space=pl.ANY),
            pl.BlockSpec((BRT, CP), lambda i: (i, 0)),
        ],
        out_specs=pl.BlockSpec((BRT, CP), lambda i: (i, 2 * D // CP)),
        out_shape=jax.ShapeDtypeStruct((M, W), jnp.float32),
        input_output_aliases={0: 0},
    )(out_sc, out_crd)


# speedup vs baseline: 1.1063x; 1.0261x over previous
import functools

import jax
import jax.numpy as jnp
from jax import lax
from jax.experimental import pallas as pl
from jax.experimental.pallas import tpu as pltpu
from jax.experimental.pallas import tpu_sc as plsc

NUM_CORES = 2
NUM_SUBCORES = 16
NW = NUM_CORES * NUM_SUBCORES
CH = 32
SUB = 8


def _window(emb, mem, q, r, bp):
    b = emb.shape[0]
    m = mem.shape[0]
    w = jnp.zeros((bp,) + emb.shape[1:], emb.dtype)
    w = lax.dynamic_update_slice(w, emb, (r,) + (0,) * (emb.ndim - 1))
    i8 = jnp.arange(SUB).reshape((SUB,) + (1,) * (emb.ndim - 1))
    head = lax.dynamic_slice_in_dim(mem, q, SUB, axis=0)
    w = w.at[0:SUB].set(jnp.where(i8 < r, head, w[0:SUB]))
    tail = mem[(q + b + jnp.arange(SUB)) % m]
    w = w.at[b:bp].set(jnp.where(i8 >= r, tail, w[b:bp]))
    return w


def kernel(mem_img, mem_gps, mem_coords, img_emb, gps_emb, gps_coords, ptr):
    M, D = mem_img.shape
    B = img_emb.shape[0]
    C = mem_coords.shape[1]
    W = 2 * D + C
    CP = 128
    BP = B + SUB
    rows_per_w = M // NW
    n_chunks = rows_per_w // CH
    n_pairs = n_chunks // 2

    p = jnp.asarray(ptr, jnp.int32) % jnp.int32(M)
    q = p & jnp.int32(-SUB)
    r = p & jnp.int32(SUB - 1)
    q_vec = jnp.full((16,), q, dtype=jnp.int32)

    ie2 = _window(img_emb, mem_img, q, r, BP)
    ge2 = _window(gps_emb, mem_gps, q, r, BP)
    gc2 = jnp.pad(_window(gps_coords, mem_coords, q, r, BP),
                  ((0, 0), (0, CP - C)))

    mesh = plsc.VectorSubcoreMesh(core_axis_name="c", subcore_axis_name="s")

    @functools.partial(
        pl.kernel,
        out_type=(jax.ShapeDtypeStruct((M, W), jnp.float32),
                  jax.ShapeDtypeStruct((M, CP), jnp.float32)),
        mesh=mesh,
        scratch_types=[
            pltpu.VMEM((CH, D), jnp.float32),
            pltpu.VMEM((CH, D), jnp.float32),
            pltpu.VMEM((CH, CP), jnp.float32),
            pltpu.VMEM((CH, D), jnp.float32),
            pltpu.VMEM((CH, D), jnp.float32),
            pltpu.VMEM((CH, CP), jnp.float32),
            pltpu.VMEM((16,), jnp.int32),
            pltpu.SemaphoreType.DMA,
            pltpu.SemaphoreType.DMA,
            pltpu.SemaphoreType.DMA,
            pltpu.SemaphoreType.DMA,
        ],
    )
    def run(mi, mg, ie, ge, gc, qv, out, ocrd,
            img0, gps0, crd0, img1, gps1, crd1, qbuf, r0, r1, w0, w1):
        wid = lax.axis_index("s") * NUM_CORES + lax.axis_index("c")
        base = pl.multiple_of(wid * rows_per_w, rows_per_w)
        pltpu.sync_copy(qv, qbuf)
        qk = qbuf[...][0]

        def read(c0, bi, bg, bc, sem):
            d = c0 - qk
            s = jnp.where(d < 0, d + M, d)
            full_in = s <= BP - CH

            @pl.when(full_in)
            def _():
                sa = pl.multiple_of(s, SUB)
                pltpu.async_copy(ie.at[pl.ds(sa, CH)], bi, sem)
                pltpu.async_copy(ge.at[pl.ds(sa, CH)], bg, sem)
                pltpu.async_copy(gc.at[pl.ds(sa, CH)], bc, sem)

            @pl.when(jnp.logical_not(full_in))
            def _():
                pltpu.async_copy(mi.at[pl.ds(c0, CH)], bi, sem)
                pltpu.async_copy(mg.at[pl.ds(c0, CH)], bg, sem)

        def wait_read(c0, bi, bg, bc, sem):
            d = c0 - qk
            s = jnp.where(d < 0, d + M, d)
            pltpu.make_async_copy(mi.at[pl.ds(0, CH)], bi, sem).wait()
            pltpu.make_async_copy(mg.at[pl.ds(0, CH)], bg, sem).wait()

            @pl.when(s <= BP - CH)
            def _():
                pltpu.make_async_copy(gc.at[pl.ds(0, CH)], bc, sem).wait()

        def overlay(c0, bi, bg, bc):
            d = c0 - qk
            s = jnp.where(d < 0, d + M, d)
            full_in = s <= BP - CH
            for k in range(CH // SUB):
                e = s + k * SUB
                e = jnp.where(e >= M, e - M, e)

                @pl.when(jnp.logical_and(jnp.logical_not(full_in), e < BP))
                def _ov(e=e, k=k):
                    ea = pl.multiple_of(e, SUB)
                    pltpu.sync_copy(ie.at[pl.ds(ea, SUB)],
                                    bi.at[pl.ds(k * SUB, SUB)])
                    pltpu.sync_copy(ge.at[pl.ds(ea, SUB)],
                                    bg.at[pl.ds(k * SUB, SUB)])
                    pltpu.sync_copy(gc.at[pl.ds(ea, SUB)],
                                    bc.at[pl.ds(k * SUB, SUB)])

        def write(c0, bi, bg, bc, sem):
            pltpu.async_copy(bi, out.at[pl.ds(c0, CH), pl.ds(0, D)], sem)
            pltpu.async_copy(bg, out.at[pl.ds(c0, CH), pl.ds(D, D)], sem)
            pltpu.async_copy(bc, ocrd.at[pl.ds(c0, CH)], sem)

        def wait_write(bi, bg, bc, sem):
            pltpu.make_async_copy(bi, out.at[pl.ds(0, CH), pl.ds(0, D)],
                                  sem).wait()
            pltpu.make_async_copy(bg, out.at[pl.ds(0, CH), pl.ds(D, D)],
                                  sem).wait()
            pltpu.make_async_copy(bc, ocrd.at[pl.ds(0, CH)], sem).wait()

        read(base, img0, gps0, crd0, r0)

        def pair_body(t2, carry):
            a = pl.multiple_of(base + (2 * t2) * CH, CH)
            b = pl.multiple_of(base + (2 * t2 + 1) * CH, CH)

            @pl.when(t2 > 0)
            def _():
                wait_write(img1, gps1, crd1, w1)

            read(b, img1, gps1, crd1, r1)
            wait_read(a, img0, gps0, crd0, r0)
            overlay(a, img0, gps0, crd0)
            write(a, img0, gps0, crd0, w0)
            wait_read(b, img1, gps1, crd1, r1)
            overlay(b, img1, gps1, crd1)
            write(b, img1, gps1, crd1, w1)

            @pl.when(t2 < n_pairs - 1)
            def _():
                wait_write(img0, gps0, crd0, w0)
                read(pl.multiple_of(base + (2 * t2 + 2) * CH, CH),
                     img0, gps0, crd0, r0)

            return carry

        lax.fori_loop(0, n_pairs, pair_body, 0)
        wait_write(img0, gps0, crd0, w0)
        wait_write(img1, gps1, crd1, w1)

    out_sc, out_crd = run(mem_img, mem_gps, ie2, ge2, gc2, q_vec)

    BRT = 512

    def tc_body(q_ref, _, crd_ref, mcr_ref, o_ref):
        i = pl.program_id(0)
        rows = lax.broadcasted_iota(jnp.int32, (BRT, 1), 0) + i * BRT
        t = rows - q_ref[0]
        t = jnp.where(t < 0, t + M, t)
        sel = t < BP
        o_ref[:, 0:C] = jnp.where(sel, crd_ref[:, 0:C], mcr_ref[...])

    return pl.pallas_call(
        tc_body,
        grid_spec=pltpu.PrefetchScalarGridSpec(
            num_scalar_prefetch=1,
            grid=(M // BRT,),
            in_specs=[
                pl.BlockSpec(memory_space=pl.ANY),
                pl.BlockSpec((BRT, CP), lambda i, qr: (i, 0)),
                pl.BlockSpec((BRT, C), lambda i, qr: (i, 0)),
            ],
            out_specs=pl.BlockSpec((BRT, CP), lambda i, qr: (i, 2 * D // CP)),
        ),
        out_shape=jax.ShapeDtypeStruct((M, W), jnp.float32),
        input_output_aliases={1: 0},
    )(q[None], out_sc, out_crd, mem_coords)
